# Initial kernel scaffold; baseline (speedup 1.0000x reference)
import functools
import jax
import jax.numpy as jnp
from jax import lax
from jax.experimental import pallas as pl
from jax.experimental.pallas import tpu as pltpu
from jax.experimental.pallas import tpu_sc as plsc

N = 10000
E = 320000
D_IN = 128
D_H = 64
D_OUT = 32

NC = 2
NS = 16
NW = NC * NS
EPW = E // NW
CHUNK = 80
NCHUNK = EPW // CHUNK
RPT = N // NS
RBLK = 125


def _spmm_sc_body(x_hbm, row_hbm, col_hbm, w_hbm, out_hbm,
                  acc, col_v, row_v, w_v, rows_v, obuf, sem):
    c = lax.axis_index("c")
    s = lax.axis_index("s")
    wid = c * NS + s

    def _zrow(r, carry):
        for j in range(D_H // 16):
            obuf[r, pl.ds(j * 16, 16)] = jnp.zeros((16,), jnp.float32)
        return carry
    lax.fori_loop(0, RBLK, _zrow, 0)
    for k in range(RPT // RBLK):
        pltpu.sync_copy(obuf, acc.at[pl.ds(s * RPT + k * RBLK, RBLK), :])
    plsc.subcore_barrier()

    ebase = wid * EPW

    def _chunk(i, carry):
        base = ebase + i * CHUNK
        pltpu.sync_copy(col_hbm.at[pl.ds(base, CHUNK)], col_v)
        pltpu.sync_copy(row_hbm.at[pl.ds(base, CHUNK)], row_v)
        pltpu.sync_copy(w_hbm.at[pl.ds(base, CHUNK)], w_v)
        pltpu.async_copy(x_hbm.at[col_v], rows_v, sem).wait()

        def _edge(e, carry2):
            wgt = w_v[e]
            for j in range(D_H // 16):
                seg = rows_v[e, pl.ds(j * 16, 16)]
                rows_v[e, pl.ds(j * 16, 16)] = seg * wgt
            return carry2
        lax.fori_loop(0, CHUNK, _edge, 0)

        pltpu.sync_copy(rows_v, acc.at[row_v], add=True)
        return carry
    lax.fori_loop(0, NCHUNK, _chunk, 0)

    plsc.subcore_barrier()

    for k in range(RPT // RBLK):
        r0 = s * RPT + k * RBLK
        pltpu.sync_copy(acc.at[pl.ds(r0, RBLK), :], obuf)
        pltpu.sync_copy(obuf, out_hbm.at[c, pl.ds(r0, RBLK), :])


_spmm_sc = pl.kernel(
    _spmm_sc_body,
    out_type=jax.ShapeDtypeStruct((NC, N, D_H), jnp.float32),
    mesh=plsc.VectorSubcoreMesh(core_axis_name="c", subcore_axis_name="s"),
    scratch_types=[
        pltpu.VMEM_SHARED((N, D_H), jnp.float32),
        pltpu.VMEM((CHUNK,), jnp.int32),
        pltpu.VMEM((CHUNK,), jnp.int32),
        pltpu.VMEM((CHUNK,), jnp.float32),
        pltpu.VMEM((CHUNK, D_H), jnp.float32),
        pltpu.VMEM((RBLK, D_H), jnp.float32),
        pltpu.SemaphoreType.DMA,
    ],
)


def _mm_body(x_ref, w_ref, o_ref):
    o_ref[...] = jnp.dot(x_ref[...], w_ref[...],
                         preferred_element_type=jnp.float32)


def _mm(x, w):
    return pl.pallas_call(
        _mm_body,
        out_shape=jax.ShapeDtypeStruct((x.shape[0], w.shape[1]), jnp.float32),
    )(x, w)


def _relu_mm_body(a_ref, b_ref, w_ref, o_ref):
    h = jnp.maximum(a_ref[...] + b_ref[...], 0.0)
    o_ref[...] = jnp.dot(h, w_ref[...], preferred_element_type=jnp.float32)


def _relu_mm(a, b, w):
    return pl.pallas_call(
        _relu_mm_body,
        out_shape=jax.ShapeDtypeStruct((a.shape[0], w.shape[1]), jnp.float32),
    )(a, b, w)


def _sample_body(a_ref, b_ref, eps_ref, zm_ref, zlv_ref, z_ref):
    s = a_ref[...] + b_ref[...]
    zm = s[:, :D_OUT]
    zlv = s[:, D_OUT:]
    zm_ref[...] = zm
    zlv_ref[...] = zlv
    z_ref[...] = zm + jnp.exp(0.5 * zlv) * eps_ref[...]


def _sample(a, b, eps):
    return pl.pallas_call(
        _sample_body,
        out_shape=(
            jax.ShapeDtypeStruct((N, D_OUT), jnp.float32),
            jax.ShapeDtypeStruct((N, D_OUT), jnp.float32),
            jax.ShapeDtypeStruct((N, D_OUT), jnp.float32),
        ),
    )(a, b, eps)


def kernel(x, edge_index, edge_weight, W0, Wm, Wv):
    row = edge_index[0]
    col = edge_index[1]
    w_cat = jnp.concatenate([Wm, Wv], axis=1)
    eps = jax.random.normal(jax.random.key(42), (N, D_OUT), dtype=jnp.float32)

    p0 = _mm(x, W0)
    s1 = _spmm_sc(p0, row, col, edge_weight)
    h1 = _relu_mm(s1[0], s1[1], w_cat)
    s2 = _spmm_sc(h1, row, col, edge_weight)
    z_mean, z_log_var, z = _sample(s2[0], s2[1], eps)
    return (z_mean, z_log_var, z)


# speedup vs baseline: 4.6808x; 4.6808x over previous
import functools
import jax
import jax.numpy as jnp
from jax import lax
from jax.experimental import pallas as pl
from jax.experimental.pallas import tpu as pltpu
from jax.experimental.pallas import tpu_sc as plsc

N = 10000
E = 320000
D_IN = 128
D_H = 64
D_OUT = 32

NC = 2
NS = 16
NW = NC * NS
EPW = E // NW
CHUNK = 80
NCHUNK = EPW // CHUNK
NPAD = 10240
RPT = NPAD // NS
RBLK = 128


def _spmm_sc_body(x_hbm, row_hbm, col_hbm, w_hbm, out_hbm,
                  acc, col_v, row_v, w_v, rows_v, obuf, sem):
    c = lax.axis_index("c")
    s = lax.axis_index("s")
    wid = c * NS + s

    def _zrow(r, carry):
        for j in range(D_H // 16):
            obuf[r, pl.ds(j * 16, 16)] = jnp.zeros((16,), jnp.float32)
        return carry
    lax.fori_loop(0, RBLK, _zrow, 0)
    for k in range(RPT // RBLK):
        pltpu.sync_copy(obuf, acc.at[pl.ds(s * RPT + k * RBLK, RBLK), :])
    plsc.subcore_barrier()

    ebase = wid * EPW

    def _chunk(i, carry):
        base = ebase + i * CHUNK
        pltpu.sync_copy(col_hbm.at[pl.ds(base, CHUNK)], col_v)
        pltpu.sync_copy(row_hbm.at[pl.ds(base, CHUNK)], row_v)
        pltpu.sync_copy(w_hbm.at[pl.ds(base, CHUNK)], w_v)
        pltpu.async_copy(x_hbm.at[col_v], rows_v, sem).wait()

        def _grp(g, carry2):
            w16 = w_v[pl.ds(g * 16, 16)]
            for j in range(16):
                e = g * 16 + j
                wgt = w16[j]
                for t in range(D_H // 16):
                    seg = rows_v[e, pl.ds(t * 16, 16)]
                    rows_v[e, pl.ds(t * 16, 16)] = seg * wgt
            return carry2
        lax.fori_loop(0, CHUNK // 16, _grp, 0)

        pltpu.sync_copy(rows_v, acc.at[row_v], add=True)
        return carry
    lax.fori_loop(0, NCHUNK, _chunk, 0)

    plsc.subcore_barrier()

    for k in range(RPT // RBLK):
        r0 = s * RPT + k * RBLK
        pltpu.sync_copy(acc.at[pl.ds(r0, RBLK), :], obuf)
        pltpu.sync_copy(obuf, out_hbm.at[c, pl.ds(r0, RBLK), :])


_spmm_sc = pl.kernel(
    _spmm_sc_body,
    out_type=jax.ShapeDtypeStruct((NC, NPAD, D_H), jnp.float32),
    mesh=plsc.VectorSubcoreMesh(core_axis_name="c", subcore_axis_name="s"),
    scratch_types=[
        pltpu.VMEM_SHARED((NPAD, D_H), jnp.float32),
        pltpu.VMEM((CHUNK,), jnp.int32),
        pltpu.VMEM((CHUNK,), jnp.int32),
        pltpu.VMEM((CHUNK,), jnp.float32),
        pltpu.VMEM((CHUNK, D_H), jnp.float32),
        pltpu.VMEM((RBLK, D_H), jnp.float32),
        pltpu.SemaphoreType.DMA,
    ],
    compiler_params=pltpu.CompilerParams(use_tc_tiling_on_sc=False),
)


def _mm_body(x_ref, w_ref, o_ref):
    o_ref[...] = jnp.dot(x_ref[...], w_ref[...],
                         preferred_element_type=jnp.float32)


def _mm(x, w):
    return pl.pallas_call(
        _mm_body,
        out_shape=jax.ShapeDtypeStruct((x.shape[0], w.shape[1]), jnp.float32),
    )(x, w)


def _relu_mm_body(a_ref, b_ref, w_ref, o_ref):
    h = jnp.maximum(a_ref[:N, :] + b_ref[:N, :], 0.0)
    o_ref[...] = jnp.dot(h, w_ref[...], preferred_element_type=jnp.float32)


def _relu_mm(a, b, w):
    return pl.pallas_call(
        _relu_mm_body,
        out_shape=jax.ShapeDtypeStruct((N, w.shape[1]), jnp.float32),
    )(a, b, w)


def _sample_body(a_ref, b_ref, eps_ref, zm_ref, zlv_ref, z_ref):
    s = a_ref[:N, :] + b_ref[:N, :]
    zm = s[:, :D_OUT]
    zlv = s[:, D_OUT:]
    zm_ref[...] = zm
    zlv_ref[...] = zlv
    z_ref[...] = zm + jnp.exp(0.5 * zlv) * eps_ref[...]


def _sample(a, b, eps):
    return pl.pallas_call(
        _sample_body,
        out_shape=(
            jax.ShapeDtypeStruct((N, D_OUT), jnp.float32),
            jax.ShapeDtypeStruct((N, D_OUT), jnp.float32),
            jax.ShapeDtypeStruct((N, D_OUT), jnp.float32),
        ),
    )(a, b, eps)


def kernel(x, edge_index, edge_weight, W0, Wm, Wv):
    row = edge_index[0]
    col = edge_index[1]
    w_cat = jnp.concatenate([Wm, Wv], axis=1)
    eps = jax.random.normal(jax.random.key(42), (N, D_OUT), dtype=jnp.float32)

    p0 = _mm(x, W0)
    s1 = _spmm_sc(p0, row, col, edge_weight)
    h1 = _relu_mm(s1[0], s1[1], w_cat)
    s2 = _spmm_sc(h1, row, col, edge_weight)
    z_mean, z_log_var, z = _sample(s2[0], s2[1], eps)
    return (z_mean, z_log_var, z)


# speedup vs baseline: 9.2481x; 1.9757x over previous
import functools
import jax
import jax.numpy as jnp
from jax import lax
from jax.experimental import pallas as pl
from jax.experimental.pallas import tpu as pltpu
from jax.experimental.pallas import tpu_sc as plsc

N = 10000
E = 320000
D_IN = 128
D_H = 64
D_OUT = 32

NC = 2
NS = 16
NW = NC * NS
EPW = E // NW
CHUNK = 80
NCHUNK = EPW // CHUNK
NPAD = 10240
RPT = NPAD // NS
RBLK = 128


def _spmm_sc_body(x_hbm, row_hbm, col_hbm, w_hbm, out_hbm,
                  acc, col_all, row_all, w_all, rv0, rv1, obuf, sem0, sem1):
    c = lax.axis_index("c")
    s = lax.axis_index("s")
    wid = c * NS + s

    pltpu.sync_copy(col_hbm.at[wid], col_all)
    pltpu.sync_copy(row_hbm.at[wid], row_all)
    pltpu.sync_copy(w_hbm.at[wid], w_all)

    def _zrow(r, carry):
        for j in range(D_H // 16):
            obuf[r, pl.ds(j * 16, 16)] = jnp.zeros((16,), jnp.float32)
        return carry
    lax.fori_loop(0, RBLK, _zrow, 0)
    for k in range(RPT // RBLK):
        pltpu.sync_copy(obuf, acc.at[pl.ds(s * RPT + k * RBLK, RBLK), :])
    plsc.subcore_barrier()

    def _process(i, rv):
        def _grp(g, carry2):
            w16 = w_all[i, pl.ds(g * 16, 16)]
            for j in range(16):
                e = g * 16 + j
                wgt = w16[j]
                for t in range(D_H // 16):
                    seg = rv[e, pl.ds(t * 16, 16)]
                    rv[e, pl.ds(t * 16, 16)] = seg * wgt
            return carry2
        lax.fori_loop(0, CHUNK // 16, _grp, 0)
        pltpu.sync_copy(rv, acc.at[row_all.at[i]], add=True)

    pltpu.async_copy(x_hbm.at[col_all.at[0]], rv0, sem0)

    def _pair(k, carry):
        i0 = 2 * k
        pltpu.async_copy(x_hbm.at[col_all.at[i0 + 1]], rv1, sem1)
        pltpu.make_async_copy(x_hbm.at[col_all.at[i0]], rv0, sem0).wait()
        _process(i0, rv0)
        pltpu.async_copy(x_hbm.at[col_all.at[i0 + 2]], rv0, sem0)
        pltpu.make_async_copy(x_hbm.at[col_all.at[i0 + 1]], rv1, sem1).wait()
        _process(i0 + 1, rv1)
        return carry
    lax.fori_loop(0, (NCHUNK - 1) // 2, _pair, 0)

    pltpu.make_async_copy(x_hbm.at[col_all.at[NCHUNK - 1]], rv0, sem0).wait()
    _process(NCHUNK - 1, rv0)

    plsc.subcore_barrier()

    for k in range(RPT // RBLK):
        r0 = s * RPT + k * RBLK
        pltpu.sync_copy(acc.at[pl.ds(r0, RBLK), :], obuf)
        pltpu.sync_copy(obuf, out_hbm.at[c, pl.ds(r0, RBLK), :])


_spmm_sc = pl.kernel(
    _spmm_sc_body,
    out_type=jax.ShapeDtypeStruct((NC, NPAD, D_H), jnp.float32),
    mesh=plsc.VectorSubcoreMesh(core_axis_name="c", subcore_axis_name="s"),
    scratch_types=[
        pltpu.VMEM_SHARED((NPAD, D_H), jnp.float32),
        pltpu.VMEM((NCHUNK, CHUNK), jnp.int32),
        pltpu.VMEM((NCHUNK, CHUNK), jnp.int32),
        pltpu.VMEM((NCHUNK, CHUNK), jnp.float32),
        pltpu.VMEM((CHUNK, D_H), jnp.float32),
        pltpu.VMEM((CHUNK, D_H), jnp.float32),
        pltpu.VMEM((RBLK, D_H), jnp.float32),
        pltpu.SemaphoreType.DMA,
        pltpu.SemaphoreType.DMA,
    ],
    compiler_params=pltpu.CompilerParams(use_tc_tiling_on_sc=False),
)


def _mm_body(x_ref, w_ref, o_ref):
    o_ref[...] = jnp.dot(x_ref[...], w_ref[...],
                         preferred_element_type=jnp.float32)


def _mm(x, w):
    return pl.pallas_call(
        _mm_body,
        out_shape=jax.ShapeDtypeStruct((x.shape[0], w.shape[1]), jnp.float32),
    )(x, w)


def _relu_mm_body(a_ref, b_ref, w_ref, o_ref):
    h = jnp.maximum(a_ref[:N, :] + b_ref[:N, :], 0.0)
    o_ref[...] = jnp.dot(h, w_ref[...], preferred_element_type=jnp.float32)


def _relu_mm(a, b, w):
    return pl.pallas_call(
        _relu_mm_body,
        out_shape=jax.ShapeDtypeStruct((N, w.shape[1]), jnp.float32),
    )(a, b, w)


def _sample_body(a_ref, b_ref, eps_ref, zm_ref, zlv_ref, z_ref):
    s = a_ref[:N, :] + b_ref[:N, :]
    zm = s[:, :D_OUT]
    zlv = s[:, D_OUT:]
    zm_ref[...] = zm
    zlv_ref[...] = zlv
    z_ref[...] = zm + jnp.exp(0.5 * zlv) * eps_ref[...]


def _sample(a, b, eps):
    return pl.pallas_call(
        _sample_body,
        out_shape=(
            jax.ShapeDtypeStruct((N, D_OUT), jnp.float32),
            jax.ShapeDtypeStruct((N, D_OUT), jnp.float32),
            jax.ShapeDtypeStruct((N, D_OUT), jnp.float32),
        ),
    )(a, b, eps)


def kernel(x, edge_index, edge_weight, W0, Wm, Wv):
    row = edge_index[0].reshape(NW, NCHUNK, CHUNK)
    col = edge_index[1].reshape(NW, NCHUNK, CHUNK)
    ew = edge_weight.reshape(NW, NCHUNK, CHUNK)
    w_cat = jnp.concatenate([Wm, Wv], axis=1)
    eps = jax.random.normal(jax.random.key(42), (N, D_OUT), dtype=jnp.float32)

    p0 = _mm(x, W0)
    s1 = _spmm_sc(p0, row, col, ew)
    h1 = _relu_mm(s1[0], s1[1], w_cat)
    s2 = _spmm_sc(h1, row, col, ew)
    z_mean, z_log_var, z = _sample(s2[0], s2[1], eps)
    return (z_mean, z_log_var, z)
